# Initial kernel scaffold; baseline (speedup 1.0000x reference)
import functools

import jax
import jax.numpy as jnp
from jax import lax
from jax.experimental import pallas as pl
from jax.experimental.pallas import tpu as pltpu
from jax.experimental.pallas import tpu_sc as plsc

NC = 2
NS = 16
LANES = 16
NW = NC * NS


def _mesh():
    return plsc.VectorSubcoreMesh(
        core_axis_name="c", subcore_axis_name="s",
        num_cores=NC, num_subcores=NS)


@functools.lru_cache(maxsize=None)
def _make_degree_kernel(N, J, C):
    RPT = N // NS

    @functools.partial(
        pl.kernel,
        mesh=_mesh(),
        out_type=jax.ShapeDtypeStruct((NC, 2, N, LANES), jnp.float32),
        scratch_types=[
            pltpu.VMEM((J, C), jnp.int32),
            pltpu.VMEM((J, C), jnp.int32),
            pltpu.VMEM((C, LANES), jnp.float32),
            pltpu.VMEM_SHARED((N, LANES), jnp.float32),
            pltpu.VMEM_SHARED((N, LANES), jnp.float32),
        ],
    )
    def deg_kernel(src_hbm, dst_hbm, zeros_hbm, ones_hbm, out_hbm,
                   src_v, dst_v, ones_v, acc_out, acc_in):
        c = lax.axis_index("c")
        s = lax.axis_index("s")
        w = c * NS + s
        r0 = s * RPT
        pltpu.sync_copy(zeros_hbm.at[pl.ds(r0, RPT)], acc_out.at[pl.ds(r0, RPT)])
        pltpu.sync_copy(zeros_hbm.at[pl.ds(r0, RPT)], acc_in.at[pl.ds(r0, RPT)])
        pltpu.sync_copy(src_hbm.at[w], src_v)
        pltpu.sync_copy(dst_hbm.at[w], dst_v)
        pltpu.sync_copy(ones_hbm, ones_v)
        plsc.subcore_barrier()

        def body(j, carry):
            pltpu.sync_copy(ones_v, acc_out.at[src_v.at[j]], add=True)
            pltpu.sync_copy(ones_v, acc_in.at[dst_v.at[j]], add=True)
            return carry

        lax.fori_loop(0, J, body, 0)
        plsc.subcore_barrier()
        pltpu.sync_copy(acc_out.at[pl.ds(r0, RPT)],
                        out_hbm.at[c, 0, pl.ds(r0, RPT)])
        pltpu.sync_copy(acc_in.at[pl.ds(r0, RPT)],
                        out_hbm.at[c, 1, pl.ds(r0, RPT)])

    return deg_kernel


@functools.lru_cache(maxsize=None)
def _make_agg_kernel(N, D, J, C):
    RPT = N // NS

    @functools.partial(
        pl.kernel,
        mesh=_mesh(),
        out_type=jax.ShapeDtypeStruct((NC, N, D), jnp.float32),
        scratch_types=[
            pltpu.VMEM((J, C), jnp.int32),
            pltpu.VMEM((J, C), jnp.int32),
            pltpu.VMEM((C, D), jnp.float32),
            pltpu.VMEM_SHARED((N, D), jnp.float32),
            pltpu.SemaphoreType.DMA,
        ],
    )
    def agg_kernel(hs_hbm, src_hbm, dst_hbm, zeros_hbm, out_hbm,
                   src_v, dst_v, rows_v, acc, sem):
        c = lax.axis_index("c")
        s = lax.axis_index("s")
        w = c * NS + s
        r0 = s * RPT
        pltpu.sync_copy(zeros_hbm.at[pl.ds(r0, RPT)], acc.at[pl.ds(r0, RPT)])
        pltpu.sync_copy(src_hbm.at[w], src_v)
        pltpu.sync_copy(dst_hbm.at[w], dst_v)
        plsc.subcore_barrier()

        def body(j, carry):
            pltpu.async_copy(hs_hbm.at[src_v.at[j]], rows_v, sem).wait()
            pltpu.sync_copy(rows_v, acc.at[dst_v.at[j]], add=True)
            return carry

        lax.fori_loop(0, J, body, 0)
        plsc.subcore_barrier()
        pltpu.sync_copy(acc.at[pl.ds(r0, RPT)], out_hbm.at[c, pl.ds(r0, RPT)])

    return agg_kernel


def _norms_body(deg_ref, x_ref, h0s_ref, nin_ref, nout_ref):
    d = deg_ref[...]
    cnt_out = jnp.sum(d[0, 0] + d[1, 0], axis=1, keepdims=True) * (1.0 / LANES)
    cnt_in = jnp.sum(d[0, 1] + d[1, 1], axis=1, keepdims=True) * (1.0 / LANES)
    nout = lax.rsqrt(1.0 + cnt_out)
    nin = lax.rsqrt(1.0 + cnt_in)
    nout_ref[...] = nout
    nin_ref[...] = nin
    h0s_ref[...] = x_ref[...] * nout


@functools.lru_cache(maxsize=None)
def _make_norms_call(N, D):
    return pl.pallas_call(
        _norms_body,
        out_shape=(
            jax.ShapeDtypeStruct((N, D), jnp.float32),
            jax.ShapeDtypeStruct((N, 1), jnp.float32),
            jax.ShapeDtypeStruct((N, 1), jnp.float32),
        ),
    )


def _layer_body(scale_out, p_ref, hs_ref, nin_ref, nout_ref, w_ref, b_ref,
                out_ref):
    pre = (p_ref[0] + p_ref[1] + hs_ref[...]) * nin_ref[...]
    h = jnp.dot(pre, w_ref[...], preferred_element_type=jnp.float32) + b_ref[...]
    if scale_out:
        h = h * nout_ref[...]
    out_ref[...] = h


@functools.lru_cache(maxsize=None)
def _make_layer_call(N, D, H, scale_out, bn=2000):
    grid = N // bn
    return pl.pallas_call(
        functools.partial(_layer_body, scale_out),
        grid=(grid,),
        in_specs=[
            pl.BlockSpec((NC, bn, D), lambda i: (0, i, 0)),
            pl.BlockSpec((bn, D), lambda i: (i, 0)),
            pl.BlockSpec((bn, 1), lambda i: (i, 0)),
            pl.BlockSpec((bn, 1), lambda i: (i, 0)),
            pl.BlockSpec((D, H), lambda i: (0, 0)),
            pl.BlockSpec((1, H), lambda i: (0, 0)),
        ],
        out_specs=pl.BlockSpec((bn, H), lambda i: (i, 0)),
        out_shape=jax.ShapeDtypeStruct((N, H), jnp.float32),
    )


def kernel(x, edge_index, W1, b1, W2, b2):
    N, D = x.shape
    H = W1.shape[1]
    E = edge_index.shape[1]
    EW = E // NW
    C = 80
    J = EW // C
    assert EW * NW == E and J * C == EW and N % NS == 0

    src3 = edge_index[0].reshape(NW, J, C)
    dst3 = edge_index[1].reshape(NW, J, C)
    zeros_d = jnp.zeros((N, D), jnp.float32)
    zeros_l = jnp.zeros((N, LANES), jnp.float32)
    ones_l = jnp.ones((C, LANES), jnp.float32)

    deg = _make_degree_kernel(N, J, C)(src3, dst3, zeros_l, ones_l)
    h0s, nin, nout = _make_norms_call(N, D)(deg, x)

    agg = _make_agg_kernel(N, D, J, C)
    p1 = agg(h0s, src3, dst3, zeros_d)
    h1s = _make_layer_call(N, D, H, True)(p1, h0s, nin, nout, W1,
                                          b1.reshape(1, H))
    p2 = agg(h1s, src3, dst3, zeros_d)
    out = _make_layer_call(N, H, H, False)(p2, h1s, nin, nout, W2,
                                           b2.reshape(1, H))
    return out


# speedup vs baseline: 10.0700x; 10.0700x over previous
import functools

import jax
import jax.numpy as jnp
from jax import lax
from jax.experimental import pallas as pl
from jax.experimental.pallas import tpu as pltpu
from jax.experimental.pallas import tpu_sc as plsc

NC = 2
NS = 16
NW = NC * NS
C = 128


def _mesh():
    return plsc.VectorSubcoreMesh(
        core_axis_name="c", subcore_axis_name="s",
        num_cores=NC, num_subcores=NS)


def _chunks(total, step):
    off = 0
    while off < total:
        yield off, min(step, total - off)
        off += step


@functools.lru_cache(maxsize=None)
def _make_degree_kernel(NP, D, CH):
    RPT = NP // NS

    @functools.partial(
        pl.kernel,
        mesh=_mesh(),
        out_type=jax.ShapeDtypeStruct((NC, NP, D), jnp.float32),
        scratch_types=[
            pltpu.VMEM((CH, C), jnp.int32),
            pltpu.VMEM((C, D), jnp.float32),
            pltpu.VMEM_SHARED((NP, D), jnp.float32),
        ],
    )
    def deg_kernel(zeros_hbm, ones_hbm, idx_hbm, out_hbm,
                   idx_v, buf_v, acc):
        c = lax.axis_index("c")
        s = lax.axis_index("s")
        r0 = pl.multiple_of(s * RPT, 8)
        pltpu.sync_copy(zeros_hbm, buf_v)
        for off, size in _chunks(RPT, C):
            rs = pl.ds(pl.multiple_of(r0 + off, 8), size)
            pltpu.sync_copy(buf_v.at[pl.ds(0, size)], acc.at[rs])
        pltpu.sync_copy(idx_hbm.at[c, s], idx_v)
        pltpu.sync_copy(ones_hbm, buf_v)
        plsc.subcore_barrier()

        @pl.loop(0, CH)
        def body(j):
            pltpu.sync_copy(buf_v, acc.at[idx_v.at[j]], add=True)

        plsc.subcore_barrier()
        for off, size in _chunks(RPT, C):
            rs = pl.ds(pl.multiple_of(r0 + off, 8), size)
            bs = pl.ds(0, size)
            pltpu.sync_copy(acc.at[rs], buf_v.at[bs])
            pltpu.sync_copy(buf_v.at[bs], out_hbm.at[c, rs])

    return deg_kernel


@functools.lru_cache(maxsize=None)
def _make_agg_kernel(NP, D, J):
    RPT = NP // NS

    @functools.partial(
        pl.kernel,
        mesh=_mesh(),
        out_type=jax.ShapeDtypeStruct((NC, NP, D), jnp.float32),
        scratch_types=[
            pltpu.VMEM((J, C), jnp.int32),
            pltpu.VMEM((J, C), jnp.int32),
            pltpu.VMEM((C, D), jnp.float32),
            pltpu.VMEM_SHARED((NP, D), jnp.float32),
            pltpu.SemaphoreType.DMA,
        ],
    )
    def agg_kernel(hs_hbm, src_hbm, dst_hbm, zeros_hbm, out_hbm,
                   src_v, dst_v, rows_v, acc, sem):
        c = lax.axis_index("c")
        s = lax.axis_index("s")
        w = c * NS + s
        r0 = pl.multiple_of(s * RPT, 8)
        pltpu.sync_copy(zeros_hbm, rows_v)
        for off, size in _chunks(RPT, C):
            rs = pl.ds(pl.multiple_of(r0 + off, 8), size)
            pltpu.sync_copy(rows_v.at[pl.ds(0, size)], acc.at[rs])
        pltpu.sync_copy(src_hbm.at[w], src_v)
        pltpu.sync_copy(dst_hbm.at[w], dst_v)
        plsc.subcore_barrier()

        @pl.loop(0, J)
        def body(j):
            pltpu.async_copy(hs_hbm.at[src_v.at[j]], rows_v, sem).wait()
            pltpu.sync_copy(rows_v, acc.at[dst_v.at[j]], add=True)

        plsc.subcore_barrier()
        for off, size in _chunks(RPT, C):
            rs = pl.ds(pl.multiple_of(r0 + off, 8), size)
            bs = pl.ds(0, size)
            pltpu.sync_copy(acc.at[rs], rows_v.at[bs])
            pltpu.sync_copy(rows_v.at[bs], out_hbm.at[c, rs])

    return agg_kernel


def _norms_body(N, NP, deg_ref, x_ref, h0s_ref, nin_ref, nout_ref):
    d = deg_ref[...]
    scale = 1.0 / d.shape[-1]
    cnt_out = jnp.sum(d[0], axis=1, keepdims=True) * scale
    cnt_in = jnp.sum(d[1], axis=1, keepdims=True) * scale
    nout = lax.rsqrt(1.0 + cnt_out)
    nin = lax.rsqrt(1.0 + cnt_in)
    nout_ref[...] = nout
    nin_ref[...] = nin
    h0s_ref[:N] = x_ref[...] * nout[:N]
    h0s_ref[N:] = jnp.zeros((NP - N, x_ref.shape[1]), jnp.float32)


@functools.lru_cache(maxsize=None)
def _make_norms_call(N, NP, D):
    return pl.pallas_call(
        functools.partial(_norms_body, N, NP),
        out_shape=(
            jax.ShapeDtypeStruct((NP, D), jnp.float32),
            jax.ShapeDtypeStruct((NP, 1), jnp.float32),
            jax.ShapeDtypeStruct((NP, 1), jnp.float32),
        ),
    )


def _layer_body(scale_out, p_ref, hs_ref, nin_ref, nout_ref, w_ref, b_ref,
                out_ref):
    pre = (p_ref[0] + p_ref[1] + hs_ref[...]) * nin_ref[...]
    h = jnp.dot(pre, w_ref[...], preferred_element_type=jnp.float32) + b_ref[...]
    if scale_out:
        h = h * nout_ref[...]
    out_ref[...] = h


@functools.lru_cache(maxsize=None)
def _make_layer_call(NR, NP, D, H, scale_out, bn):
    grid = NR // bn
    return pl.pallas_call(
        functools.partial(_layer_body, scale_out),
        grid=(grid,),
        in_specs=[
            pl.BlockSpec((NC, bn, D), lambda i: (0, i, 0)),
            pl.BlockSpec((bn, D), lambda i: (i, 0)),
            pl.BlockSpec((bn, 1), lambda i: (i, 0)),
            pl.BlockSpec((bn, 1), lambda i: (i, 0)),
            pl.BlockSpec((D, H), lambda i: (0, 0)),
            pl.BlockSpec((1, H), lambda i: (0, 0)),
        ],
        out_specs=pl.BlockSpec((bn, H), lambda i: (i, 0)),
        out_shape=jax.ShapeDtypeStruct((NR, H), jnp.float32),
    )


def kernel(x, edge_index, W1, b1, W2, b2):
    N, D = x.shape
    H = W1.shape[1]
    E = edge_index.shape[1]
    NP = -(-N // (NS * 8)) * (NS * 8)
    J = -(-E // (NW * C))
    EP = NW * J * C
    CH = EP // (NS * C)
    assert N % 8 == 0 and NP > N

    pad = N + (jnp.arange(EP - E, dtype=jnp.int32) % (NP - N))
    srcp = jnp.concatenate([edge_index[0], pad])
    dstp = jnp.concatenate([edge_index[1], pad])
    src3 = srcp.reshape(NW, J, C)
    dst3 = dstp.reshape(NW, J, C)
    idx4 = jnp.stack([srcp, dstp]).reshape(NC, NS, CH, C)
    zeros_d = jnp.zeros((C, D), jnp.float32)
    ones_d = jnp.ones((C, D), jnp.float32)

    deg = _make_degree_kernel(NP, D, CH)(zeros_d, ones_d, idx4)
    h0s, nin, nout = _make_norms_call(N, NP, D)(deg, x)

    agg = _make_agg_kernel(NP, D, J)
    p1 = agg(h0s, src3, dst3, zeros_d)
    h1s = _make_layer_call(NP, NP, D, H, True, 632)(p1, h0s, nin, nout, W1,
                                                    b1.reshape(1, H))
    p2 = agg(h1s, src3, dst3, zeros_d)
    out = _make_layer_call(N, NP, H, H, False, 2000)(p2, h1s, nin, nout, W2,
                                                     b2.reshape(1, H))
    return out


# speedup vs baseline: 13.0876x; 1.2997x over previous
import functools

import jax
import jax.numpy as jnp
from jax import lax
from jax.experimental import pallas as pl
from jax.experimental.pallas import tpu as pltpu
from jax.experimental.pallas import tpu_sc as plsc

NC = 2
NS = 16
NW = NC * NS
C = 128


def _mesh():
    return plsc.VectorSubcoreMesh(
        core_axis_name="c", subcore_axis_name="s",
        num_cores=NC, num_subcores=NS)


def _chunks(total, step):
    off = 0
    while off < total:
        yield off, min(step, total - off)
        off += step


@functools.lru_cache(maxsize=None)
def _make_degree_kernel(NP, D, CH):
    RPT = NP // NS

    @functools.partial(
        pl.kernel,
        mesh=_mesh(),
        out_type=jax.ShapeDtypeStruct((NC, NP, D), jnp.float32),
        scratch_types=[
            pltpu.VMEM((CH, C), jnp.int32),
            pltpu.VMEM((C, D), jnp.float32),
            pltpu.VMEM_SHARED((NP, D), jnp.float32),
        ],
    )
    def deg_kernel(zeros_hbm, ones_hbm, idx_hbm, out_hbm,
                   idx_v, buf_v, acc):
        c = lax.axis_index("c")
        s = lax.axis_index("s")
        r0 = pl.multiple_of(s * RPT, 8)
        pltpu.sync_copy(zeros_hbm, buf_v)
        for off, size in _chunks(RPT, C):
            rs = pl.ds(pl.multiple_of(r0 + off, 8), size)
            pltpu.sync_copy(buf_v.at[pl.ds(0, size)], acc.at[rs])
        pltpu.sync_copy(idx_hbm.at[c, s], idx_v)
        pltpu.sync_copy(ones_hbm, buf_v)
        plsc.subcore_barrier()

        @pl.loop(0, CH)
        def body(j):
            pltpu.sync_copy(buf_v, acc.at[idx_v.at[j]], add=True)

        plsc.subcore_barrier()
        for off, size in _chunks(RPT, C):
            rs = pl.ds(pl.multiple_of(r0 + off, 8), size)
            bs = pl.ds(0, size)
            pltpu.sync_copy(acc.at[rs], buf_v.at[bs])
            pltpu.sync_copy(buf_v.at[bs], out_hbm.at[c, rs])

    return deg_kernel


@functools.lru_cache(maxsize=None)
def _make_agg_kernel(NP, D, J, NPH):
    RPT = NP // NS
    PH = J // NPH
    assert PH % 2 == 0 and PH >= 4

    @functools.partial(
        pl.kernel,
        mesh=_mesh(),
        out_type=jax.ShapeDtypeStruct((NC, NP, D), jnp.float32),
        scratch_types=[
            pltpu.VMEM((PH, C), jnp.int32),
            pltpu.VMEM((PH, C), jnp.int32),
            pltpu.VMEM((C, D), jnp.float32),
            pltpu.VMEM((C, D), jnp.float32),
            pltpu.VMEM_SHARED((NP, D), jnp.float32),
            pltpu.SemaphoreType.DMA,
            pltpu.SemaphoreType.DMA,
        ],
    )
    def agg_kernel(hs_hbm, src_hbm, dst_hbm, zeros_hbm, out_hbm,
                   src_v, dst_v, rows0, rows1, acc, sem0, sem1):
        c = lax.axis_index("c")
        s = lax.axis_index("s")
        w = c * NS + s
        r0 = pl.multiple_of(s * RPT, 8)
        pltpu.sync_copy(zeros_hbm, rows0)
        for off, size in _chunks(RPT, C):
            rs = pl.ds(pl.multiple_of(r0 + off, 8), size)
            pltpu.sync_copy(rows0.at[pl.ds(0, size)], acc.at[rs])
        plsc.subcore_barrier()

        def gather(j, rows, sem):
            pltpu.async_copy(hs_hbm.at[src_v.at[j]], rows, sem)

        def gwait(j, rows, sem):
            pltpu.make_async_copy(hs_hbm.at[src_v.at[j]], rows, sem).wait()

        def scat(j, rows):
            pltpu.sync_copy(rows, acc.at[dst_v.at[j]], add=True)

        @pl.loop(0, NPH)
        def phase(p):
            pltpu.sync_copy(src_hbm.at[w, pl.ds(p * PH, PH)], src_v)
            pltpu.sync_copy(dst_hbm.at[w, pl.ds(p * PH, PH)], dst_v)
            gather(0, rows0, sem0)

            @pl.loop(0, PH // 2 - 1)
            def body(jj):
                j0 = jj * 2
                gather(j0 + 1, rows1, sem1)
                gwait(j0, rows0, sem0)
                scat(j0, rows0)
                gather(j0 + 2, rows0, sem0)
                gwait(j0 + 1, rows1, sem1)
                scat(j0 + 1, rows1)

            gather(PH - 1, rows1, sem1)
            gwait(PH - 2, rows0, sem0)
            scat(PH - 2, rows0)
            gwait(PH - 1, rows1, sem1)
            scat(PH - 1, rows1)

        plsc.subcore_barrier()
        for off, size in _chunks(RPT, C):
            rs = pl.ds(pl.multiple_of(r0 + off, 8), size)
            bs = pl.ds(0, size)
            pltpu.sync_copy(acc.at[rs], rows0.at[bs])
            pltpu.sync_copy(rows0.at[bs], out_hbm.at[c, rs])

    return agg_kernel


def _norms_body(N, NP, deg_ref, x_ref, h0s_ref, nin_ref, nout_ref):
    d = deg_ref[...]
    scale = 1.0 / d.shape[-1]
    cnt_out = jnp.sum(d[0], axis=1, keepdims=True) * scale
    cnt_in = jnp.sum(d[1], axis=1, keepdims=True) * scale
    nout = lax.rsqrt(1.0 + cnt_out)
    nin = lax.rsqrt(1.0 + cnt_in)
    nout_ref[...] = nout
    nin_ref[...] = nin
    h0s_ref[:N] = x_ref[...] * nout[:N]
    h0s_ref[N:] = jnp.zeros((NP - N, x_ref.shape[1]), jnp.float32)


@functools.lru_cache(maxsize=None)
def _make_norms_call(N, NP, D):
    return pl.pallas_call(
        functools.partial(_norms_body, N, NP),
        out_shape=(
            jax.ShapeDtypeStruct((NP, D), jnp.float32),
            jax.ShapeDtypeStruct((NP, 1), jnp.float32),
            jax.ShapeDtypeStruct((NP, 1), jnp.float32),
        ),
    )


def _layer_body(scale_out, p_ref, hs_ref, nin_ref, nout_ref, w_ref, b_ref,
                out_ref):
    pre = (p_ref[0] + p_ref[1] + hs_ref[...]) * nin_ref[...]
    h = jnp.dot(pre, w_ref[...], preferred_element_type=jnp.float32) + b_ref[...]
    if scale_out:
        h = h * nout_ref[...]
    out_ref[...] = h


@functools.lru_cache(maxsize=None)
def _make_layer_call(NR, NP, D, H, scale_out, bn):
    grid = NR // bn
    return pl.pallas_call(
        functools.partial(_layer_body, scale_out),
        grid=(grid,),
        in_specs=[
            pl.BlockSpec((NC, bn, D), lambda i: (0, i, 0)),
            pl.BlockSpec((bn, D), lambda i: (i, 0)),
            pl.BlockSpec((bn, 1), lambda i: (i, 0)),
            pl.BlockSpec((bn, 1), lambda i: (i, 0)),
            pl.BlockSpec((D, H), lambda i: (0, 0)),
            pl.BlockSpec((1, H), lambda i: (0, 0)),
        ],
        out_specs=pl.BlockSpec((bn, H), lambda i: (i, 0)),
        out_shape=jax.ShapeDtypeStruct((NR, H), jnp.float32),
    )


def kernel(x, edge_index, W1, b1, W2, b2):
    N, D = x.shape
    H = W1.shape[1]
    E = edge_index.shape[1]
    NP = -(-N // (NS * 8)) * (NS * 8)
    J = (-(-E // (NW * C)) + 3) // 4 * 4
    EP = NW * J * C
    CH = EP // (NS * C)
    assert N % 8 == 0 and NP > N

    pad = N + (jnp.arange(EP - E, dtype=jnp.int32) % (NP - N))
    srcp = jnp.concatenate([edge_index[0], pad])
    dstp = jnp.concatenate([edge_index[1], pad])
    src3 = srcp.reshape(NW, J, C)
    dst3 = dstp.reshape(NW, J, C)
    idx4 = jnp.stack([srcp, dstp]).reshape(NC, NS, CH, C)
    zeros_d = jnp.zeros((C, D), jnp.float32)
    ones_d = jnp.ones((C, D), jnp.float32)

    deg = _make_degree_kernel(NP, D, CH)(zeros_d, ones_d, idx4)
    h0s, nin, nout = _make_norms_call(N, NP, D)(deg, x)

    agg = _make_agg_kernel(NP, D, J, 2)
    p1 = agg(h0s, src3, dst3, zeros_d)
    h1s = _make_layer_call(NP, NP, D, H, True, 632)(p1, h0s, nin, nout, W1,
                                                    b1.reshape(1, H))
    p2 = agg(h1s, src3, dst3, zeros_d)
    out = _make_layer_call(N, NP, H, H, False, 2000)(p2, h1s, nin, nout, W2,
                                                     b2.reshape(1, H))
    return out


# speedup vs baseline: 13.1393x; 1.0040x over previous
import functools

import jax
import jax.numpy as jnp
from jax import lax
from jax.experimental import pallas as pl
from jax.experimental.pallas import tpu as pltpu
from jax.experimental.pallas import tpu_sc as plsc

NC = 2
NS = 16
NW = NC * NS
C = 128


def _mesh():
    return plsc.VectorSubcoreMesh(
        core_axis_name="c", subcore_axis_name="s",
        num_cores=NC, num_subcores=NS)


def _chunks(total, step):
    off = 0
    while off < total:
        yield off, min(step, total - off)
        off += step


@functools.lru_cache(maxsize=None)
def _make_degree_kernel(NP, D, CH):
    RPT = NP // NS

    @functools.partial(
        pl.kernel,
        mesh=_mesh(),
        out_type=jax.ShapeDtypeStruct((NC, NP, D), jnp.float32),
        scratch_types=[
            pltpu.VMEM((CH, C), jnp.int32),
            pltpu.VMEM((C, D), jnp.float32),
            pltpu.VMEM_SHARED((NP, D), jnp.float32),
            pltpu.SemaphoreType.DMA,
        ],
    )
    def deg_kernel(zeros_hbm, ones_hbm, idx_hbm, out_hbm,
                   idx_v, buf_v, acc, sem):
        c = lax.axis_index("c")
        s = lax.axis_index("s")
        r0 = pl.multiple_of(s * RPT, 8)
        pltpu.sync_copy(zeros_hbm, buf_v)
        for off, size in _chunks(RPT, C):
            rs = pl.ds(pl.multiple_of(r0 + off, 8), size)
            pltpu.sync_copy(buf_v.at[pl.ds(0, size)], acc.at[rs])
        pltpu.sync_copy(idx_hbm.at[c, s], idx_v)
        pltpu.sync_copy(ones_hbm, buf_v)
        plsc.subcore_barrier()

        DEPTH = 4

        def fire(j):
            pltpu.async_copy(buf_v, acc.at[idx_v.at[j]], sem, add=True)

        def drain(j):
            pltpu.make_async_copy(buf_v, acc.at[idx_v.at[j]], sem).wait()

        for j in range(DEPTH):
            fire(j)

        @pl.loop(DEPTH, CH)
        def body(j):
            drain(j - DEPTH)
            fire(j)

        for j in range(DEPTH):
            drain(CH - DEPTH + j)

        plsc.subcore_barrier()
        for off, size in _chunks(RPT, C):
            rs = pl.ds(pl.multiple_of(r0 + off, 8), size)
            bs = pl.ds(0, size)
            pltpu.sync_copy(acc.at[rs], buf_v.at[bs])
            pltpu.sync_copy(buf_v.at[bs], out_hbm.at[c, rs])

    return deg_kernel


@functools.lru_cache(maxsize=None)
def _make_agg_kernel(NP, D, J, NPH):
    RPT = NP // NS
    PH = J // NPH
    assert PH % 2 == 0 and PH >= 4

    @functools.partial(
        pl.kernel,
        mesh=_mesh(),
        out_type=jax.ShapeDtypeStruct((NC, NP, D), jnp.float32),
        scratch_types=[
            pltpu.VMEM((PH, C), jnp.int32),
            pltpu.VMEM((PH, C), jnp.int32),
            pltpu.VMEM((C, D), jnp.float32),
            pltpu.VMEM((C, D), jnp.float32),
            pltpu.VMEM_SHARED((NP, D), jnp.float32),
            pltpu.SemaphoreType.DMA,
            pltpu.SemaphoreType.DMA,
        ],
    )
    def agg_kernel(hs_hbm, src_hbm, dst_hbm, zeros_hbm, out_hbm,
                   src_v, dst_v, rows0, rows1, acc, sem0, sem1):
        c = lax.axis_index("c")
        s = lax.axis_index("s")
        w = c * NS + s
        r0 = pl.multiple_of(s * RPT, 8)
        pltpu.sync_copy(zeros_hbm, rows0)
        for off, size in _chunks(RPT, C):
            rs = pl.ds(pl.multiple_of(r0 + off, 8), size)
            pltpu.sync_copy(rows0.at[pl.ds(0, size)], acc.at[rs])
        plsc.subcore_barrier()

        def gather(j, rows, sem):
            pltpu.async_copy(hs_hbm.at[src_v.at[j]], rows, sem)

        def gwait(j, rows, sem):
            pltpu.make_async_copy(hs_hbm.at[src_v.at[j]], rows, sem).wait()

        def scat(j, rows):
            pltpu.sync_copy(rows, acc.at[dst_v.at[j]], add=True)

        @pl.loop(0, NPH)
        def phase(p):
            pltpu.sync_copy(src_hbm.at[w, pl.ds(p * PH, PH)], src_v)
            pltpu.sync_copy(dst_hbm.at[w, pl.ds(p * PH, PH)], dst_v)
            gather(0, rows0, sem0)

            @pl.loop(0, PH // 2 - 1)
            def body(jj):
                j0 = jj * 2
                gather(j0 + 1, rows1, sem1)
                gwait(j0, rows0, sem0)
                scat(j0, rows0)
                gather(j0 + 2, rows0, sem0)
                gwait(j0 + 1, rows1, sem1)
                scat(j0 + 1, rows1)

            gather(PH - 1, rows1, sem1)
            gwait(PH - 2, rows0, sem0)
            scat(PH - 2, rows0)
            gwait(PH - 1, rows1, sem1)
            scat(PH - 1, rows1)

        plsc.subcore_barrier()
        for off, size in _chunks(RPT, C):
            rs = pl.ds(pl.multiple_of(r0 + off, 8), size)
            bs = pl.ds(0, size)
            pltpu.sync_copy(acc.at[rs], rows0.at[bs])
            pltpu.sync_copy(rows0.at[bs], out_hbm.at[c, rs])

    return agg_kernel


def _norms_body(N, NP, deg_ref, x_ref, h0s_ref, nin_ref, nout_ref):
    d = deg_ref[...]
    scale = 1.0 / d.shape[-1]
    cnt_out = jnp.sum(d[0], axis=1, keepdims=True) * scale
    cnt_in = jnp.sum(d[1], axis=1, keepdims=True) * scale
    nout = lax.rsqrt(1.0 + cnt_out)
    nin = lax.rsqrt(1.0 + cnt_in)
    nout_ref[...] = nout
    nin_ref[...] = nin
    h0s_ref[:N] = x_ref[...] * nout[:N]
    h0s_ref[N:] = jnp.zeros((NP - N, x_ref.shape[1]), jnp.float32)


@functools.lru_cache(maxsize=None)
def _make_norms_call(N, NP, D):
    return pl.pallas_call(
        functools.partial(_norms_body, N, NP),
        out_shape=(
            jax.ShapeDtypeStruct((NP, D), jnp.float32),
            jax.ShapeDtypeStruct((NP, 1), jnp.float32),
            jax.ShapeDtypeStruct((NP, 1), jnp.float32),
        ),
    )


def _layer_body(scale_out, p_ref, hs_ref, nin_ref, nout_ref, w_ref, b_ref,
                out_ref):
    pre = (p_ref[0] + p_ref[1] + hs_ref[...]) * nin_ref[...]
    h = jnp.dot(pre, w_ref[...], preferred_element_type=jnp.float32) + b_ref[...]
    if scale_out:
        h = h * nout_ref[...]
    out_ref[...] = h


@functools.lru_cache(maxsize=None)
def _make_layer_call(NR, NP, D, H, scale_out, bn):
    grid = NR // bn
    return pl.pallas_call(
        functools.partial(_layer_body, scale_out),
        grid=(grid,),
        in_specs=[
            pl.BlockSpec((NC, bn, D), lambda i: (0, i, 0)),
            pl.BlockSpec((bn, D), lambda i: (i, 0)),
            pl.BlockSpec((bn, 1), lambda i: (i, 0)),
            pl.BlockSpec((bn, 1), lambda i: (i, 0)),
            pl.BlockSpec((D, H), lambda i: (0, 0)),
            pl.BlockSpec((1, H), lambda i: (0, 0)),
        ],
        out_specs=pl.BlockSpec((bn, H), lambda i: (i, 0)),
        out_shape=jax.ShapeDtypeStruct((NR, H), jnp.float32),
    )


def kernel(x, edge_index, W1, b1, W2, b2):
    N, D = x.shape
    H = W1.shape[1]
    E = edge_index.shape[1]
    NP = -(-N // (NS * 8)) * (NS * 8)
    J = (-(-E // (NW * C)) + 3) // 4 * 4
    EP = NW * J * C
    CH = EP // (NS * C)
    assert N % 8 == 0 and NP > N

    pad = N + (jnp.arange(EP - E, dtype=jnp.int32) % (NP - N))
    srcp = jnp.concatenate([edge_index[0], pad])
    dstp = jnp.concatenate([edge_index[1], pad])
    src3 = srcp.reshape(NW, J, C)
    dst3 = dstp.reshape(NW, J, C)
    idx4 = jnp.stack([srcp, dstp]).reshape(NC, NS, CH, C)
    zeros_d = jnp.zeros((C, D), jnp.float32)
    ones_d = jnp.ones((C, D), jnp.float32)

    deg = _make_degree_kernel(NP, D, CH)(zeros_d, ones_d, idx4)
    h0s, nin, nout = _make_norms_call(N, NP, D)(deg, x)

    agg = _make_agg_kernel(NP, D, J, 2)
    p1 = agg(h0s, src3, dst3, zeros_d)
    h1s = _make_layer_call(NP, NP, D, H, True, 632)(p1, h0s, nin, nout, W1,
                                                    b1.reshape(1, H))
    p2 = agg(h1s, src3, dst3, zeros_d)
    out = _make_layer_call(N, NP, H, H, False, 2000)(p2, h1s, nin, nout, W2,
                                                     b2.reshape(1, H))
    return out
